# baseline (speedup 1.0000x reference)
import functools

import jax
import jax.numpy as jnp
from jax import lax
from jax.experimental import pallas as pl
from jax.experimental.pallas import tpu as pltpu
from jax.experimental.pallas import tpu_sc as plsc

NC, NS = 2, 16
NW = NC * NS
N = 10000
E = 320000
D = 128
CH = 128
NCHUNK = 80
DSLAB = 40
EPT = CH * NCHUNK
E_PAD = EPT * NW
N_PAD = 10240
RPT = N_PAD // NS


def _mm_body(x_ref, w_ref, o_ref):
    o_ref[...] = jnp.dot(x_ref[...], w_ref[...],
                         preferred_element_type=jnp.float32)


def _matmul(x, w, bm=1000):
    m, k = x.shape
    n = w.shape[1]
    return pl.pallas_call(
        _mm_body,
        grid=(m // bm,),
        in_specs=[pl.BlockSpec((bm, k), lambda i: (i, 0)),
                  pl.BlockSpec((k, n), lambda i: (0, 0))],
        out_specs=pl.BlockSpec((bm, n), lambda i: (i, 0)),
        out_shape=jax.ShapeDtypeStruct((m, n), jnp.float32),
    )(x, w)


def _relu_body(p_ref, b_ref, o_ref):
    o_ref[...] = jnp.maximum(p_ref[0] + p_ref[1] + b_ref[...], 0.0)


def _combine_relu(p, b, bm=1000):
    return pl.pallas_call(
        _relu_body,
        grid=(N // bm,),
        in_specs=[pl.BlockSpec((2, bm, D), lambda i: (0, i, 0)),
                  pl.BlockSpec((1, D), lambda i: (0, 0))],
        out_specs=pl.BlockSpec((bm, D), lambda i: (i, 0)),
        out_shape=jax.ShapeDtypeStruct((N, D), jnp.float32),
    )(p, b.reshape(1, D))


def _final_body(q_ref, w_ref, b_ref, o_ref):
    s = q_ref[0] + q_ref[1]
    o_ref[...] = jnp.dot(s, w_ref[...],
                         preferred_element_type=jnp.float32) + b_ref[...]


def _final(q, w, b, bm=1000):
    n_out = w.shape[1]
    return pl.pallas_call(
        _final_body,
        grid=(N // bm,),
        in_specs=[pl.BlockSpec((2, bm, D), lambda i: (0, i, 0)),
                  pl.BlockSpec((D, n_out), lambda i: (0, 0)),
                  pl.BlockSpec((1, n_out), lambda i: (0, 0))],
        out_specs=pl.BlockSpec((bm, n_out), lambda i: (i, 0)),
        out_shape=jax.ShapeDtypeStruct((N, n_out), jnp.float32),
    )(q, w, b.reshape(1, n_out))


def _agg_body(h_ref, src_ref, dst_ref, out_ref,
              src_v, dst_v, rows_v, agg_sh, gsem, ssem):
    cid = lax.axis_index("c")
    sid = lax.axis_index("s")
    wid = cid * NS + sid

    for r in range(CH):
        for c8 in range(D // 16):
            rows_v[0, r, pl.ds(c8 * 16, 16)] = jnp.zeros((16,), jnp.float32)

    def zloop(k, carry):
        pltpu.sync_copy(rows_v.at[0],
                        agg_sh.at[pl.ds(sid * RPT + k * CH, CH)])
        return carry
    lax.fori_loop(0, RPT // CH, zloop, 0)
    plsc.subcore_barrier()

    pltpu.sync_copy(src_ref.at[wid], src_v)

    HH = CH // 2

    def _gather(j, b):
        for half in range(2):
            pltpu.async_copy(
                h_ref.at[src_v.at[j, pl.ds(half * HH, HH)]],
                rows_v.at[b, pl.ds(half * HH, HH)], gsem)

    def _gather_wait(j, b):
        for half in range(2):
            pltpu.make_async_copy(
                h_ref.at[src_v.at[j, pl.ds(half * HH, HH)]],
                rows_v.at[b, pl.ds(half * HH, HH)], gsem).wait()

    _gather(0, 0)

    def eloop(j, carry):
        b = lax.rem(j, 2)
        rem = lax.rem(j, DSLAB)

        @pl.when(rem == 0)
        def _stage_dst():
            slab = j // DSLAB
            pltpu.sync_copy(dst_ref.at[wid, pl.ds(slab * DSLAB, DSLAB)],
                            dst_v)

        _gather_wait(j, b)
        pltpu.async_copy(rows_v.at[b], agg_sh.at[dst_v.at[rem]], ssem,
                         add=True)

        @pl.when(j >= 1)
        def _free_other():
            pltpu.make_async_copy(h_ref.at[pl.ds(0, CH)], rows_v.at[1 - b],
                                  ssem).wait()

        @pl.when(j < NCHUNK - 1)
        def _prefetch():
            jn = lax.min(j + 1, NCHUNK - 1)
            _gather(jn, 1 - b)

        return carry
    lax.fori_loop(0, NCHUNK, eloop, 0)
    pltpu.make_async_copy(h_ref.at[pl.ds(0, CH)], rows_v.at[0], ssem).wait()
    plsc.subcore_barrier()

    def oloop(m, carry):
        b = lax.rem(m, 2)

        @pl.when(m >= 2)
        def _free_buf():
            pltpu.make_async_copy(h_ref.at[pl.ds(0, CH)], rows_v.at[b],
                                  ssem).wait()

        base = sid * RPT + m * CH
        pltpu.sync_copy(agg_sh.at[pl.ds(base, CH)], rows_v.at[b])
        pltpu.async_copy(rows_v.at[b], out_ref.at[cid, pl.ds(base, CH)],
                         ssem)
        return carry
    lax.fori_loop(0, RPT // CH, oloop, 0)
    pltpu.make_async_copy(h_ref.at[pl.ds(0, CH)], rows_v.at[0], ssem).wait()
    pltpu.make_async_copy(h_ref.at[pl.ds(0, CH)], rows_v.at[1], ssem).wait()


_agg = functools.partial(
    pl.kernel,
    out_type=jax.ShapeDtypeStruct((NC, N_PAD, D), jnp.float32),
    mesh=plsc.VectorSubcoreMesh(core_axis_name="c", subcore_axis_name="s",
                                num_cores=NC, num_subcores=NS),
    scratch_types=[
        pltpu.VMEM((NCHUNK, CH), jnp.int32),
        pltpu.VMEM((DSLAB, CH), jnp.int32),
        pltpu.VMEM((2, CH, D), jnp.float32),
        pltpu.VMEM_SHARED((N_PAD, D), jnp.float32),
        pltpu.SemaphoreType.DMA,
        pltpu.SemaphoreType.DMA,
    ],
)(_agg_body)


def kernel(x, edge_index, W1, b1, W2, b2):
    src = edge_index[0]
    dst = edge_index[1]
    pad = E_PAD - E
    pad_src = jnp.arange(pad, dtype=jnp.int32) * 41 % N
    pad_dst = N + jnp.arange(pad, dtype=jnp.int32) % (N_PAD - N)
    srcp = jnp.concatenate([src, pad_src]).reshape(NW, NCHUNK, CH)
    dstp = jnp.concatenate([dst, pad_dst]).reshape(NW, NCHUNK, CH)

    h = _matmul(x, W1)
    p = _agg(h, srcp, dstp)
    hr = _combine_relu(p, b1)
    q = _agg(hr, srcp, dstp)
    return _final(q, W2, b2)


# speedup vs baseline: 1.0064x; 1.0064x over previous
import functools

import jax
import jax.numpy as jnp
import numpy as np
from jax import lax
from jax.experimental import pallas as pl
from jax.experimental.pallas import tpu as pltpu
from jax.experimental.pallas import tpu_sc as plsc

NC, NS = 2, 16
NW = NC * NS
N = 10000
E = 320000
D = 128
CH = 128
NCHUNK = 80
DSLAB = 40
EPT = CH * NCHUNK
E_PAD = EPT * NW
N_PAD = 10240
RPT = N_PAD // NS


def _mm_body(x_ref, w_ref, o_ref):
    o_ref[...] = jnp.dot(x_ref[...], w_ref[...],
                         preferred_element_type=jnp.float32)


def _matmul(x, w, bm=1000):
    m, k = x.shape
    n = w.shape[1]
    return pl.pallas_call(
        _mm_body,
        grid=(m // bm,),
        in_specs=[pl.BlockSpec((bm, k), lambda i: (i, 0)),
                  pl.BlockSpec((k, n), lambda i: (0, 0))],
        out_specs=pl.BlockSpec((bm, n), lambda i: (i, 0)),
        out_shape=jax.ShapeDtypeStruct((m, n), jnp.float32),
    )(x, w)


def _relu_body(p_ref, b_ref, o_ref):
    o_ref[...] = jnp.maximum(p_ref[0] + p_ref[1] + b_ref[...], 0.0)


def _combine_relu(p, b, bm=1000):
    return pl.pallas_call(
        _relu_body,
        grid=(N // bm,),
        in_specs=[pl.BlockSpec((2, bm, D), lambda i: (0, i, 0)),
                  pl.BlockSpec((1, D), lambda i: (0, 0))],
        out_specs=pl.BlockSpec((bm, D), lambda i: (i, 0)),
        out_shape=jax.ShapeDtypeStruct((N, D), jnp.float32),
    )(p, b.reshape(1, D))


def _final_body(q_ref, w_ref, b_ref, o_ref):
    s = q_ref[0] + q_ref[1]
    o_ref[...] = jnp.dot(s, w_ref[...],
                         preferred_element_type=jnp.float32) + b_ref[...]


def _final(q, w, b, bm=1000):
    n_out = w.shape[1]
    return pl.pallas_call(
        _final_body,
        grid=(N // bm,),
        in_specs=[pl.BlockSpec((2, bm, D), lambda i: (0, i, 0)),
                  pl.BlockSpec((D, n_out), lambda i: (0, 0)),
                  pl.BlockSpec((1, n_out), lambda i: (0, 0))],
        out_specs=pl.BlockSpec((bm, n_out), lambda i: (i, 0)),
        out_shape=jax.ShapeDtypeStruct((N, n_out), jnp.float32),
    )(q, w, b.reshape(1, n_out))


def _agg_body(h_ref, src_ref, dst_ref, out_ref,
              src_v, dst_v, rows_v, agg_sh, gsem, ssem):
    cid = lax.axis_index("c")
    sid = lax.axis_index("s")
    wid = cid * NS + sid

    HH = CH // 2

    def _gather(j, b):
        for half in range(2):
            pltpu.async_copy(
                h_ref.at[src_v.at[j, pl.ds(half * HH, HH)]],
                rows_v.at[b, pl.ds(half * HH, HH)], gsem)

    def _gather_wait(j, b):
        for half in range(2):
            pltpu.make_async_copy(
                h_ref.at[src_v.at[j, pl.ds(half * HH, HH)]],
                rows_v.at[b, pl.ds(half * HH, HH)], gsem).wait()

    pltpu.sync_copy(src_ref.at[wid], src_v)
    _gather(0, 0)

    for r in range(CH):
        for c8 in range(D // 16):
            rows_v[1, r, pl.ds(c8 * 16, 16)] = jnp.zeros((16,), jnp.float32)

    def zloop(k, carry):
        pltpu.sync_copy(rows_v.at[1],
                        agg_sh.at[pl.ds(sid * RPT + k * CH, CH)])
        return carry
    lax.fori_loop(0, RPT // CH, zloop, 0)
    plsc.subcore_barrier()


    def eloop(j, carry):
        b = lax.rem(j, 2)
        rem = lax.rem(j, DSLAB)

        @pl.when(rem == 0)
        def _stage_dst():
            slab = j // DSLAB
            pltpu.sync_copy(dst_ref.at[wid, pl.ds(slab * DSLAB, DSLAB)],
                            dst_v)

        _gather_wait(j, b)
        pltpu.async_copy(rows_v.at[b], agg_sh.at[dst_v.at[rem]], ssem,
                         add=True)

        @pl.when(j >= 1)
        def _free_other():
            pltpu.make_async_copy(h_ref.at[pl.ds(0, CH)], rows_v.at[1 - b],
                                  ssem).wait()

        @pl.when(j < NCHUNK - 1)
        def _prefetch():
            jn = lax.min(j + 1, NCHUNK - 1)
            _gather(jn, 1 - b)

        return carry
    lax.fori_loop(0, NCHUNK, eloop, 0)
    pltpu.make_async_copy(h_ref.at[pl.ds(0, CH)], rows_v.at[0], ssem).wait()
    plsc.subcore_barrier()

    def oloop(m, carry):
        b = lax.rem(m, 2)

        @pl.when(m >= 2)
        def _free_buf():
            pltpu.make_async_copy(h_ref.at[pl.ds(0, CH)], rows_v.at[b],
                                  ssem).wait()

        base = sid * RPT + m * CH
        pltpu.sync_copy(agg_sh.at[pl.ds(base, CH)], rows_v.at[b])
        pltpu.async_copy(rows_v.at[b], out_ref.at[cid, pl.ds(base, CH)],
                         ssem)
        return carry
    lax.fori_loop(0, RPT // CH, oloop, 0)
    pltpu.make_async_copy(h_ref.at[pl.ds(0, CH)], rows_v.at[0], ssem).wait()
    pltpu.make_async_copy(h_ref.at[pl.ds(0, CH)], rows_v.at[1], ssem).wait()


_agg = functools.partial(
    pl.kernel,
    out_type=jax.ShapeDtypeStruct((NC, N_PAD, D), jnp.float32),
    mesh=plsc.VectorSubcoreMesh(core_axis_name="c", subcore_axis_name="s",
                                num_cores=NC, num_subcores=NS),
    scratch_types=[
        pltpu.VMEM((NCHUNK, CH), jnp.int32),
        pltpu.VMEM((DSLAB, CH), jnp.int32),
        pltpu.VMEM((2, CH, D), jnp.float32),
        pltpu.VMEM_SHARED((N_PAD, D), jnp.float32),
        pltpu.SemaphoreType.DMA,
        pltpu.SemaphoreType.DMA,
    ],
)(_agg_body)


_PAD_SRC = np.arange(E_PAD - E, dtype=np.int32) * 41 % N
_PAD_DST = (N + np.arange(E_PAD - E, dtype=np.int32) % (N_PAD - N)).astype(
    np.int32)

def kernel(x, edge_index, W1, b1, W2, b2):
    src = edge_index[0]
    dst = edge_index[1]
    srcp = jnp.concatenate([src, _PAD_SRC]).reshape(NW, NCHUNK, CH)
    dstp = jnp.concatenate([dst, _PAD_DST]).reshape(NW, NCHUNK, CH)

    h = _matmul(x, W1)
    p = _agg(h, srcp, dstp)
    hr = _combine_relu(p, b1)
    q = _agg(hr, srcp, dstp)
    return _final(q, W2, b2)


# speedup vs baseline: 1.0244x; 1.0179x over previous
import functools

import jax
import jax.numpy as jnp
import numpy as np
from jax import lax
from jax.experimental import pallas as pl
from jax.experimental.pallas import tpu as pltpu
from jax.experimental.pallas import tpu_sc as plsc

NC, NS = 2, 16
NW = NC * NS
N = 10000
E = 320000
D = 128
CH = 128
NCHUNK = 80
DSLAB = 40
EPT = CH * NCHUNK
E_PAD = EPT * NW
N_PAD = 10240
RPT = N_PAD // NS


def _mm_body(x_ref, w_ref, o_ref):
    o_ref[...] = jnp.dot(x_ref[...], w_ref[...],
                         preferred_element_type=jnp.float32)


def _matmul(x, w, bm=1000):
    m, k = x.shape
    n = w.shape[1]
    return pl.pallas_call(
        _mm_body,
        grid=(m // bm,),
        in_specs=[pl.BlockSpec((bm, k), lambda i: (i, 0)),
                  pl.BlockSpec((k, n), lambda i: (0, 0))],
        out_specs=pl.BlockSpec((bm, n), lambda i: (i, 0)),
        out_shape=jax.ShapeDtypeStruct((m, n), jnp.float32),
    )(x, w)


def _relu_body(p_ref, b_ref, o_ref):
    o_ref[...] = jnp.maximum(p_ref[0] + p_ref[1] + b_ref[...], 0.0)


def _combine_relu(p, b, bm=1000):
    return pl.pallas_call(
        _relu_body,
        grid=(N // bm,),
        in_specs=[pl.BlockSpec((2, bm, D), lambda i: (0, i, 0)),
                  pl.BlockSpec((1, D), lambda i: (0, 0))],
        out_specs=pl.BlockSpec((bm, D), lambda i: (i, 0)),
        out_shape=jax.ShapeDtypeStruct((N, D), jnp.float32),
    )(p, b.reshape(1, D))


def _final_body(q_ref, w_ref, b_ref, o_ref):
    s = q_ref[0] + q_ref[1]
    o_ref[...] = jnp.dot(s, w_ref[...],
                         preferred_element_type=jnp.float32) + b_ref[...]


def _final(q, w, b, bm=1000):
    n_out = w.shape[1]
    return pl.pallas_call(
        _final_body,
        grid=(N // bm,),
        in_specs=[pl.BlockSpec((2, bm, D), lambda i: (0, i, 0)),
                  pl.BlockSpec((D, n_out), lambda i: (0, 0)),
                  pl.BlockSpec((1, n_out), lambda i: (0, 0))],
        out_specs=pl.BlockSpec((bm, n_out), lambda i: (i, 0)),
        out_shape=jax.ShapeDtypeStruct((N, n_out), jnp.float32),
    )(q, w, b.reshape(1, n_out))


ERM = (NW - 1) * NCHUNK
EM = ERM * CH


def _agg_body(h_ref, src_ref, dst_ref, tsrc_ref, tdst_ref, out_ref,
              src_v, dst_v, rows_v, agg_sh, gsem, ssem):
    cid = lax.axis_index("c")
    sid = lax.axis_index("s")
    wid = cid * NS + sid

    HH = CH // 2

    def _gather(j, b):
        for half in range(2):
            pltpu.async_copy(
                h_ref.at[src_v.at[j, pl.ds(half * HH, HH)]],
                rows_v.at[b, pl.ds(half * HH, HH)], gsem)

    def _gather_wait(j, b):
        for half in range(2):
            pltpu.make_async_copy(
                h_ref.at[src_v.at[j, pl.ds(half * HH, HH)]],
                rows_v.at[b, pl.ds(half * HH, HH)], gsem).wait()

    base = wid * NCHUNK

    @pl.when(wid < NW - 1)
    def _stage_src():
        pltpu.sync_copy(src_ref.at[pl.ds(base, NCHUNK)], src_v)

    @pl.when(wid == NW - 1)
    def _stage_src_last():
        pltpu.sync_copy(tsrc_ref, src_v)

    _gather(0, 0)

    for r in range(CH):
        for c8 in range(D // 16):
            rows_v[1, r, pl.ds(c8 * 16, 16)] = jnp.zeros((16,), jnp.float32)

    def zloop(k, carry):
        pltpu.sync_copy(rows_v.at[1],
                        agg_sh.at[pl.ds(sid * RPT + k * CH, CH)])
        return carry
    lax.fori_loop(0, RPT // CH, zloop, 0)
    plsc.subcore_barrier()


    def eloop(j, carry):
        b = lax.rem(j, 2)
        rem = lax.rem(j, DSLAB)

        @pl.when(rem == 0)
        def _stage_dst():
            slab = j // DSLAB

            @pl.when(wid < NW - 1)
            def _plain():
                pltpu.sync_copy(
                    dst_ref.at[pl.ds(base + slab * DSLAB, DSLAB)], dst_v)

            @pl.when(wid == NW - 1)
            def _last():
                pltpu.sync_copy(tdst_ref.at[pl.ds(slab * DSLAB, DSLAB)],
                                dst_v)

        _gather_wait(j, b)
        pltpu.async_copy(rows_v.at[b], agg_sh.at[dst_v.at[rem]], ssem,
                         add=True)

        @pl.when(j >= 1)
        def _free_other():
            pltpu.make_async_copy(h_ref.at[pl.ds(0, CH)], rows_v.at[1 - b],
                                  ssem).wait()

        @pl.when(j < NCHUNK - 1)
        def _prefetch():
            jn = lax.min(j + 1, NCHUNK - 1)
            _gather(jn, 1 - b)

        return carry
    lax.fori_loop(0, NCHUNK, eloop, 0)
    pltpu.make_async_copy(h_ref.at[pl.ds(0, CH)], rows_v.at[0], ssem).wait()
    plsc.subcore_barrier()

    def oloop(m, carry):
        b = lax.rem(m, 2)

        @pl.when(m >= 2)
        def _free_buf():
            pltpu.make_async_copy(h_ref.at[pl.ds(0, CH)], rows_v.at[b],
                                  ssem).wait()

        base = sid * RPT + m * CH
        pltpu.sync_copy(agg_sh.at[pl.ds(base, CH)], rows_v.at[b])
        pltpu.async_copy(rows_v.at[b], out_ref.at[cid, pl.ds(base, CH)],
                         ssem)
        return carry
    lax.fori_loop(0, RPT // CH, oloop, 0)
    pltpu.make_async_copy(h_ref.at[pl.ds(0, CH)], rows_v.at[0], ssem).wait()
    pltpu.make_async_copy(h_ref.at[pl.ds(0, CH)], rows_v.at[1], ssem).wait()


_agg = functools.partial(
    pl.kernel,
    out_type=jax.ShapeDtypeStruct((NC, N_PAD, D), jnp.float32),
    mesh=plsc.VectorSubcoreMesh(core_axis_name="c", subcore_axis_name="s",
                                num_cores=NC, num_subcores=NS),
    scratch_types=[
        pltpu.VMEM((NCHUNK, CH), jnp.int32),
        pltpu.VMEM((DSLAB, CH), jnp.int32),
        pltpu.VMEM((2, CH, D), jnp.float32),
        pltpu.VMEM_SHARED((N_PAD, D), jnp.float32),
        pltpu.SemaphoreType.DMA,
        pltpu.SemaphoreType.DMA,
    ],
)(_agg_body)


_PAD_SRC = np.arange(E_PAD - E, dtype=np.int32) * 41 % N
_PAD_DST = (N + np.arange(E_PAD - E, dtype=np.int32) % (N_PAD - N)).astype(
    np.int32)

def kernel(x, edge_index, W1, b1, W2, b2):
    srcp = edge_index[0, :EM].reshape(ERM, CH)
    dstp = edge_index[1, :EM].reshape(ERM, CH)
    tsrc = jnp.concatenate(
        [edge_index[0, EM:], jnp.asarray(_PAD_SRC)]).reshape(NCHUNK, CH)
    tdst = jnp.concatenate(
        [edge_index[1, EM:], jnp.asarray(_PAD_DST)]).reshape(NCHUNK, CH)

    h = _matmul(x, W1)
    p = _agg(h, srcp, dstp, tsrc, tdst)
    hr = _combine_relu(p, b1)
    q = _agg(hr, srcp, dstp, tsrc, tdst)
    return _final(q, W2, b2)


# speedup vs baseline: 1.0525x; 1.0275x over previous
import functools

import jax
import jax.numpy as jnp
import numpy as np
from jax import lax
from jax.experimental import pallas as pl
from jax.experimental.pallas import tpu as pltpu
from jax.experimental.pallas import tpu_sc as plsc

NC, NS = 2, 16
NW = NC * NS
N = 10000
E = 320000
D = 128
CH = 128
NCHUNK = 80
DSLAB = 40
EPT = CH * NCHUNK
E_PAD = EPT * NW
N_PAD = 10240
RPT = N_PAD // NS


def _mm_body(x_ref, w_ref, o_ref):
    o_ref[...] = jnp.dot(x_ref[...], w_ref[...],
                         preferred_element_type=jnp.float32)


def _matmul(x, w, bm=2000):
    m, k = x.shape
    n = w.shape[1]
    return pl.pallas_call(
        _mm_body,
        grid=(m // bm,),
        in_specs=[pl.BlockSpec((bm, k), lambda i: (i, 0)),
                  pl.BlockSpec((k, n), lambda i: (0, 0))],
        out_specs=pl.BlockSpec((bm, n), lambda i: (i, 0)),
        out_shape=jax.ShapeDtypeStruct((m, n), jnp.float32),
    )(x, w)


def _relu_body(p_ref, b_ref, o_ref):
    o_ref[...] = jnp.maximum(p_ref[0] + p_ref[1] + b_ref[...], 0.0)


def _combine_relu(p, b, bm=2000):
    return pl.pallas_call(
        _relu_body,
        grid=(N // bm,),
        in_specs=[pl.BlockSpec((2, bm, D), lambda i: (0, i, 0)),
                  pl.BlockSpec((1, D), lambda i: (0, 0))],
        out_specs=pl.BlockSpec((bm, D), lambda i: (i, 0)),
        out_shape=jax.ShapeDtypeStruct((N, D), jnp.float32),
    )(p, b.reshape(1, D))


def _final_body(q_ref, w_ref, b_ref, o_ref):
    s = q_ref[0] + q_ref[1]
    o_ref[...] = jnp.dot(s, w_ref[...],
                         preferred_element_type=jnp.float32) + b_ref[...]


def _final(q, w, b, bm=2000):
    n_out = w.shape[1]
    return pl.pallas_call(
        _final_body,
        grid=(N // bm,),
        in_specs=[pl.BlockSpec((2, bm, D), lambda i: (0, i, 0)),
                  pl.BlockSpec((D, n_out), lambda i: (0, 0)),
                  pl.BlockSpec((1, n_out), lambda i: (0, 0))],
        out_specs=pl.BlockSpec((bm, n_out), lambda i: (i, 0)),
        out_shape=jax.ShapeDtypeStruct((N, n_out), jnp.float32),
    )(q, w, b.reshape(1, n_out))


ERM = (NW - 1) * NCHUNK
EM = ERM * CH


def _agg_body(h_ref, src_ref, dst_ref, tsrc_ref, tdst_ref, out_ref,
              src_v, dst_v, rows_v, agg_sh, gsem, ssem):
    cid = lax.axis_index("c")
    sid = lax.axis_index("s")
    wid = cid * NS + sid

    HH = CH // 2

    def _gather(j, b):
        for half in range(2):
            pltpu.async_copy(
                h_ref.at[src_v.at[j, pl.ds(half * HH, HH)]],
                rows_v.at[b, pl.ds(half * HH, HH)], gsem)

    def _gather_wait(j, b):
        for half in range(2):
            pltpu.make_async_copy(
                h_ref.at[src_v.at[j, pl.ds(half * HH, HH)]],
                rows_v.at[b, pl.ds(half * HH, HH)], gsem).wait()

    base = wid * NCHUNK

    @pl.when(wid < NW - 1)
    def _stage_src():
        pltpu.sync_copy(src_ref.at[pl.ds(base, NCHUNK)], src_v)

    @pl.when(wid == NW - 1)
    def _stage_src_last():
        pltpu.sync_copy(tsrc_ref, src_v)

    _gather(0, 0)

    for r in range(CH):
        for c8 in range(D // 16):
            rows_v[1, r, pl.ds(c8 * 16, 16)] = jnp.zeros((16,), jnp.float32)

    def zloop(k, carry):
        pltpu.sync_copy(rows_v.at[1],
                        agg_sh.at[pl.ds(sid * RPT + k * CH, CH)])
        return carry
    lax.fori_loop(0, RPT // CH, zloop, 0)
    plsc.subcore_barrier()


    def eloop(j, carry):
        b = lax.rem(j, 2)
        rem = lax.rem(j, DSLAB)

        @pl.when(rem == 0)
        def _stage_dst():
            slab = j // DSLAB

            @pl.when(wid < NW - 1)
            def _plain():
                pltpu.sync_copy(
                    dst_ref.at[pl.ds(base + slab * DSLAB, DSLAB)], dst_v)

            @pl.when(wid == NW - 1)
            def _last():
                pltpu.sync_copy(tdst_ref.at[pl.ds(slab * DSLAB, DSLAB)],
                                dst_v)

        _gather_wait(j, b)
        pltpu.async_copy(rows_v.at[b], agg_sh.at[dst_v.at[rem]], ssem,
                         add=True)

        @pl.when(j >= 1)
        def _free_other():
            pltpu.make_async_copy(h_ref.at[pl.ds(0, CH)], rows_v.at[1 - b],
                                  ssem).wait()

        @pl.when(j < NCHUNK - 1)
        def _prefetch():
            jn = lax.min(j + 1, NCHUNK - 1)
            _gather(jn, 1 - b)

        return carry
    lax.fori_loop(0, NCHUNK, eloop, 0)
    pltpu.make_async_copy(h_ref.at[pl.ds(0, CH)], rows_v.at[0], ssem).wait()
    plsc.subcore_barrier()

    def oloop(m, carry):
        b = lax.rem(m, 2)

        @pl.when(m >= 2)
        def _free_buf():
            pltpu.make_async_copy(h_ref.at[pl.ds(0, CH)], rows_v.at[b],
                                  ssem).wait()

        base = sid * RPT + m * CH
        pltpu.sync_copy(agg_sh.at[pl.ds(base, CH)], rows_v.at[b])
        pltpu.async_copy(rows_v.at[b], out_ref.at[cid, pl.ds(base, CH)],
                         ssem)
        return carry
    lax.fori_loop(0, RPT // CH, oloop, 0)
    pltpu.make_async_copy(h_ref.at[pl.ds(0, CH)], rows_v.at[0], ssem).wait()
    pltpu.make_async_copy(h_ref.at[pl.ds(0, CH)], rows_v.at[1], ssem).wait()


_agg = functools.partial(
    pl.kernel,
    out_type=jax.ShapeDtypeStruct((NC, N_PAD, D), jnp.float32),
    mesh=plsc.VectorSubcoreMesh(core_axis_name="c", subcore_axis_name="s",
                                num_cores=NC, num_subcores=NS),
    scratch_types=[
        pltpu.VMEM((NCHUNK, CH), jnp.int32),
        pltpu.VMEM((DSLAB, CH), jnp.int32),
        pltpu.VMEM((2, CH, D), jnp.float32),
        pltpu.VMEM_SHARED((N_PAD, D), jnp.float32),
        pltpu.SemaphoreType.DMA,
        pltpu.SemaphoreType.DMA,
    ],
)(_agg_body)


_PAD_SRC = np.arange(E_PAD - E, dtype=np.int32) * 41 % N
_PAD_DST = (N + np.arange(E_PAD - E, dtype=np.int32) % (N_PAD - N)).astype(
    np.int32)

def kernel(x, edge_index, W1, b1, W2, b2):
    srcp = edge_index[0, :EM].reshape(ERM, CH)
    dstp = edge_index[1, :EM].reshape(ERM, CH)
    tsrc = jnp.concatenate(
        [edge_index[0, EM:], jnp.asarray(_PAD_SRC)]).reshape(NCHUNK, CH)
    tdst = jnp.concatenate(
        [edge_index[1, EM:], jnp.asarray(_PAD_DST)]).reshape(NCHUNK, CH)

    h = _matmul(x, W1)
    p = _agg(h, srcp, dstp, tsrc, tdst)
    hr = _combine_relu(p, b1)
    q = _agg(hr, srcp, dstp, tsrc, tdst)
    return _final(q, W2, b2)
